# baseline (speedup 1.0000x reference)
import functools

import jax
import jax.numpy as jnp
from jax import lax
from jax.experimental import pallas as pl
from jax.experimental.pallas import tpu as pltpu
from jax.experimental.pallas import tpu_sc as plsc

_NC = 50
_PW = 5.0
_QP = 304


def _dense_body(x_ref, pp_ref, lab_ref, tp_ref, c_ref, w_ref, p_ref, s_ref,
                xl_scr, ppt_scr):
    B, Q, C = x_ref.shape
    T = lab_ref.shape[1]
    D = pp_ref.shape[2]

    x = x_ref[...]
    m = jnp.max(x, axis=2, keepdims=True)
    e = jnp.exp(x - m)

    labs = lab_ref[...]
    ci = lax.broadcasted_iota(jnp.int32, (B, T + 2, C), 2)
    ri = lax.broadcasted_iota(jnp.int32, (B, T + 2, C), 1)
    oh_lab = (ci == jnp.pad(labs, ((0, 0), (0, 2)))[:, :, None]).astype(jnp.float32)
    oh_bg = (ci == _NC).astype(jnp.float32)
    sel = jnp.where(ri < T, oh_lab, jnp.where(ri == T, 1.0, oh_bg))
    for b in range(B):
        xl_scr[b] = lax.dot_general(sel[b], e[b], (((1,), (1,)), ((), ())),
                                    preferred_element_type=jnp.float32)
    exl = xl_scr[:, :T, :]
    s_row = xl_scr[:, T:T + 1, :]
    ebg = xl_scr[:, T + 1:T + 2, :]

    cls_cost = -(exl / s_row)
    logls = jnp.log(s_row)
    lp = jnp.log(exl) - logls
    bg_lp = jnp.log(ebg) - logls
    ce_bg = -jnp.sum(bg_lp)

    di = lax.broadcasted_iota(jnp.int32, (D, D), 0)
    dj = lax.broadcasted_iota(jnp.int32, (D, D), 1)
    ident = (di == dj).astype(jnp.float32)
    pp = pp_ref[...]
    for b in range(B):
        ppt_scr[b] = lax.dot_general(ident, pp[b], (((1,), (1,)), ((), ())),
                                     preferred_element_type=jnp.float32)

    tp = tp_ref[...]
    poly = jnp.zeros((B, T, Q), dtype=jnp.float32)
    for d in range(D):
        a_d = ppt_scr[:, d:d + 1, :]
        b_d = tp[:, :, d:d + 1]
        poly = poly + jnp.abs(a_d - b_d)

    c_ref[:, :, :Q] = cls_cost + _PW * poly
    c_ref[:, :, Q:] = jnp.full((B, T, _QP - Q), jnp.inf, jnp.float32)
    p_ref[:, :, :Q] = poly
    p_ref[:, :, Q:] = jnp.zeros((B, T, _QP - Q), jnp.float32)
    w_ref[:, :, :Q] = bg_lp - lp
    w_ref[:, :, Q:] = jnp.zeros((B, T, _QP - Q), jnp.float32)

    lane = lax.broadcasted_iota(jnp.int32, (1, 2), 1)
    s_ref[...] = jnp.where(lane == 0, ce_bg, 0.0)


def _sc_match(cmat, wmat, pmat):
    B, T, QP = cmat.shape
    nj = QP // 16
    nw = T * QP
    mesh = plsc.VectorSubcoreMesh(core_axis_name="c", subcore_axis_name="s")
    cmat = cmat.reshape(B * T * QP)
    wmat = wmat.reshape(B * T * QP)
    pmat = pmat.reshape(B * T * QP)

    @functools.partial(
        pl.kernel, mesh=mesh,
        compiler_params=pltpu.CompilerParams(needs_layout_passes=False),
        out_type=jax.ShapeDtypeStruct((B * 16,), jnp.float32),
        scratch_types=[
            pltpu.VMEM((T * QP,), jnp.float32),
            pltpu.VMEM((T * QP,), jnp.float32),
            pltpu.VMEM((T * QP,), jnp.float32),
            pltpu.VMEM((QP,), jnp.float32),
            pltpu.VMEM((16,), jnp.float32),
        ],
    )
    def k(c_hbm, w_hbm, p_hbm, out_hbm, c_v, w_v, p_v, used_v, out_v):
        cid = lax.axis_index("c")
        sid = lax.axis_index("s")
        w = sid * 2 + cid

        @pl.when(w < B)
        def _():
            pltpu.sync_copy(c_hbm.at[pl.ds(w * nw, nw)], c_v)
            pltpu.sync_copy(w_hbm.at[pl.ds(w * nw, nw)], w_v)
            pltpu.sync_copy(p_hbm.at[pl.ds(w * nw, nw)], p_v)
            il = lax.iota(jnp.int32, 16)
            for j in range(nj):
                used_v[pl.ds(16 * j, 16)] = jnp.zeros((16,), jnp.float32)

            def gstep(t, carry):
                pacc, cacc = carry
                base = QP * t
                bestv = jnp.full((16,), jnp.inf, jnp.float32)
                besti = jnp.full((16,), jnp.int32(2 ** 30))
                for j in range(nj):
                    cv = c_v[pl.ds(base + 16 * j, 16)]
                    uv = used_v[pl.ds(16 * j, 16)]
                    mv = cv + uv
                    upd = mv < bestv
                    bestv = jnp.where(upd, mv, bestv)
                    besti = jnp.where(upd, 16 * j + il, besti)
                mn = jnp.min(bestv)
                cand = jnp.where(bestv == mn, besti, jnp.int32(2 ** 30))
                i = jnp.min(cand)
                jstar = i // 16
                sel = il == (i - 16 * jstar)
                ustar = used_v[pl.ds(16 * jstar, 16)]
                used_v[pl.ds(16 * jstar, 16)] = jnp.where(
                    sel, jnp.float32(jnp.inf), ustar)
                pacc = pacc + jnp.where(
                    sel, p_v[pl.ds(base + 16 * jstar, 16)], 0.0)
                cacc = cacc + jnp.where(
                    sel, w_v[pl.ds(base + 16 * jstar, 16)], 0.0)
                return pacc, cacc

            z16 = jnp.zeros((16,), jnp.float32)
            pacc, cacc = lax.fori_loop(0, T, gstep, (z16, z16))
            psum = jnp.sum(pacc)
            csum = jnp.sum(cacc)
            out_v[...] = jnp.where(il == 0, psum,
                                   jnp.where(il == 1, csum, 0.0))
            pltpu.sync_copy(out_v, out_hbm.at[pl.ds(w * 16, 16)])

    return k(cmat, wmat, pmat)


def kernel(pred_logits, pred_polylines, tgt_labels, tgt_polylines):
    B, Q, C = pred_logits.shape
    T = tgt_labels.shape[1]
    cmat, wmat, pmat, sc_bg = pl.pallas_call(
        _dense_body,
        out_shape=[
            jax.ShapeDtypeStruct((B, T, _QP), jnp.float32),
            jax.ShapeDtypeStruct((B, T, _QP), jnp.float32),
            jax.ShapeDtypeStruct((B, T, _QP), jnp.float32),
            jax.ShapeDtypeStruct((1, 2), jnp.float32),
        ],
        scratch_shapes=[
            pltpu.VMEM((B, T + 2, Q), jnp.float32),
            pltpu.VMEM((B, pred_polylines.shape[2], Q), jnp.float32),
        ],
    )(pred_logits, pred_polylines, tgt_labels.astype(jnp.int32), tgt_polylines)
    parts = _sc_match(cmat, wmat, pmat).reshape(B, 16)
    loss_ce = (sc_bg[0, 0] + jnp.sum(parts[:, 1])) / jnp.float32(B * Q)
    loss_poly = jnp.sum(parts[:, 0]) / jnp.float32(B * T)
    return jnp.stack([loss_ce, loss_poly])


# speedup vs baseline: 1.0347x; 1.0347x over previous
import functools

import jax
import jax.numpy as jnp
from jax import lax
from jax.experimental import pallas as pl
from jax.experimental.pallas import tpu as pltpu
from jax.experimental.pallas import tpu_sc as plsc

_NC = 50
_PW = 5.0
_QP = 304


def _dense_body(x_ref, pp_ref, lab_ref, tp_ref, big_ref, bg_ref, s_ref,
                xl_scr, ppt_scr):
    B, Q, C = x_ref.shape
    T = lab_ref.shape[1]
    D = pp_ref.shape[2]

    x = x_ref[...]
    m = jnp.max(x, axis=2, keepdims=True)
    e = jnp.exp(x - m)

    labs = lab_ref[...]
    ci = lax.broadcasted_iota(jnp.int32, (B, T + 2, C), 2)
    ri = lax.broadcasted_iota(jnp.int32, (B, T + 2, C), 1)
    oh_lab = (ci == jnp.pad(labs, ((0, 0), (0, 2)))[:, :, None]).astype(jnp.float32)
    oh_bg = (ci == _NC).astype(jnp.float32)
    sel = jnp.where(ri < T, oh_lab, jnp.where(ri == T, 1.0, oh_bg))
    for b in range(B):
        xl_scr[b] = lax.dot_general(sel[b], e[b], (((1,), (1,)), ((), ())),
                                    preferred_element_type=jnp.float32)
    exl = xl_scr[:, :T, :]
    s_row = xl_scr[:, T:T + 1, :]
    ebg = xl_scr[:, T + 1:T + 2, :]

    cls_cost = -(exl / s_row)
    logls = jnp.log(s_row)
    lp = jnp.log(exl) - logls
    bg_lp = jnp.log(ebg) - logls
    ce_bg = -jnp.sum(bg_lp)

    di = lax.broadcasted_iota(jnp.int32, (D, D), 0)
    dj = lax.broadcasted_iota(jnp.int32, (D, D), 1)
    ident = (di == dj).astype(jnp.float32)
    pp = pp_ref[...]
    for b in range(B):
        ppt_scr[b] = lax.dot_general(ident, pp[b], (((1,), (1,)), ((), ())),
                                     preferred_element_type=jnp.float32)

    tp = tp_ref[...]
    poly = jnp.zeros((B, T, Q), dtype=jnp.float32)
    for d in range(D):
        a_d = ppt_scr[:, d:d + 1, :]
        b_d = tp[:, :, d:d + 1]
        poly = poly + jnp.abs(a_d - b_d)

    big_ref[0, :, :, :Q] = cls_cost + _PW * poly
    big_ref[0, :, :, Q:] = jnp.full((B, T, _QP - Q), jnp.inf, jnp.float32)
    big_ref[1, :, :, :Q] = lp
    big_ref[1, :, :, Q:] = jnp.zeros((B, T, _QP - Q), jnp.float32)
    bg_ref[:, :1, :Q] = bg_lp
    bg_ref[:, :1, Q:] = jnp.zeros((B, 1, _QP - Q), jnp.float32)

    lane = lax.broadcasted_iota(jnp.int32, (1, 2), 1)
    s_ref[...] = jnp.where(lane == 0, ce_bg, 0.0)


def _sc_match(big, bg):
    _, B, T, QP = big.shape
    nj = QP // 16
    nw = T * QP
    mesh = plsc.VectorSubcoreMesh(core_axis_name="c", subcore_axis_name="s")
    big = big.reshape(2 * B * T * QP)
    bg = bg.reshape(B * QP)

    @functools.partial(
        pl.kernel, mesh=mesh,
        compiler_params=pltpu.CompilerParams(needs_layout_passes=False),
        out_type=jax.ShapeDtypeStruct((B * 16,), jnp.float32),
        scratch_types=[
            pltpu.VMEM((T * QP,), jnp.float32),
            pltpu.VMEM((T * QP,), jnp.float32),
            pltpu.VMEM((QP,), jnp.float32),
            pltpu.VMEM((QP,), jnp.float32),
            pltpu.VMEM((16,), jnp.float32),
        ],
    )
    def k(big_hbm, bg_hbm, out_hbm, c_v, lp_v, bg_v, used_v, out_v):
        cid = lax.axis_index("c")
        sid = lax.axis_index("s")
        w = sid * 2 + cid

        @pl.when(w < B)
        def _():
            pltpu.sync_copy(big_hbm.at[pl.ds(w * nw, nw)], c_v)
            pltpu.sync_copy(big_hbm.at[pl.ds(B * nw + w * nw, nw)], lp_v)
            pltpu.sync_copy(bg_hbm.at[pl.ds(w * QP, QP)], bg_v)
            il = lax.iota(jnp.int32, 16)
            for j in range(nj):
                used_v[pl.ds(16 * j, 16)] = jnp.zeros((16,), jnp.float32)

            def gstep(t, carry):
                pacc, cacc = carry
                base = QP * t
                bestv = jnp.full((16,), jnp.inf, jnp.float32)
                besti = jnp.full((16,), jnp.int32(2 ** 30))
                for j in range(nj):
                    cv = c_v[pl.ds(base + 16 * j, 16)]
                    uv = used_v[pl.ds(16 * j, 16)]
                    mv = cv + uv
                    upd = mv < bestv
                    bestv = jnp.where(upd, mv, bestv)
                    besti = jnp.where(upd, 16 * j + il, besti)
                mn = jnp.min(bestv)
                cand = jnp.where(bestv == mn, besti, jnp.int32(2 ** 30))
                i = jnp.min(cand)
                jstar = i // 16
                sel = il == (i - 16 * jstar)
                ustar = used_v[pl.ds(16 * jstar, 16)]
                used_v[pl.ds(16 * jstar, 16)] = jnp.where(
                    sel, jnp.float32(jnp.inf), ustar)
                lpc = lp_v[pl.ds(base + 16 * jstar, 16)]
                bgc = bg_v[pl.ds(16 * jstar, 16)]
                pacc = pacc + jnp.where(
                    sel, (mn + jnp.exp(lpc)) * jnp.float32(1.0 / _PW), 0.0)
                cacc = cacc + jnp.where(sel, bgc - lpc, 0.0)
                return pacc, cacc

            z16 = jnp.zeros((16,), jnp.float32)
            pacc, cacc = lax.fori_loop(0, T, gstep, (z16, z16))
            psum = jnp.sum(pacc)
            csum = jnp.sum(cacc)
            out_v[...] = jnp.where(il == 0, psum,
                                   jnp.where(il == 1, csum, 0.0))
            pltpu.sync_copy(out_v, out_hbm.at[pl.ds(w * 16, 16)])

    return k(big, bg)


def kernel(pred_logits, pred_polylines, tgt_labels, tgt_polylines):
    B, Q, C = pred_logits.shape
    T = tgt_labels.shape[1]
    big, bgm, sc_bg = pl.pallas_call(
        _dense_body,
        out_shape=[
            jax.ShapeDtypeStruct((2, B, T, _QP), jnp.float32),
            jax.ShapeDtypeStruct((B, 1, _QP), jnp.float32),
            jax.ShapeDtypeStruct((1, 2), jnp.float32),
        ],
        scratch_shapes=[
            pltpu.VMEM((B, T + 2, Q), jnp.float32),
            pltpu.VMEM((B, pred_polylines.shape[2], Q), jnp.float32),
        ],
    )(pred_logits, pred_polylines, tgt_labels.astype(jnp.int32), tgt_polylines)
    parts = _sc_match(big, bgm).reshape(B, 16)
    loss_ce = (sc_bg[0, 0] + jnp.sum(parts[:, 1])) / jnp.float32(B * Q)
    loss_poly = jnp.sum(parts[:, 0]) / jnp.float32(B * T)
    return jnp.stack([loss_ce, loss_poly])
